# baseline (speedup 1.0000x reference)
import functools

import jax
import jax.numpy as jnp
from jax import lax
from jax.experimental import pallas as pl
from jax.experimental.pallas import tpu as pltpu
from jax.experimental.pallas import tpu_sc as plsc

_ROWS = 4
_SPLIT = 128


def _make_gather(B, H, D, n_workers):
    rows_per_w = B // n_workers
    n_iters = rows_per_w // _ROWS
    assert n_iters % 2 == 0
    mesh = plsc.VectorSubcoreMesh(core_axis_name="c", subcore_axis_name="s")
    nc = plsc.get_sparse_core_info().num_cores

    @functools.partial(
        pl.kernel,
        mesh=mesh,
        out_type=jax.ShapeDtypeStruct((B, H, D), jnp.float32),
        scratch_types=[
            pltpu.VMEM((_ROWS, H), jnp.int32),
            pltpu.VMEM((_ROWS, H), jnp.int32),
            pltpu.VMEM((_ROWS, H, D), jnp.float32),
            pltpu.VMEM((_ROWS, H, D), jnp.float32),
            pltpu.SemaphoreType.DMA,
            pltpu.SemaphoreType.DMA,
            pltpu.SemaphoreType.DMA,
            pltpu.SemaphoreType.DMA,
        ],
        compiler_params=pltpu.CompilerParams(use_tc_tiling_on_sc=False),
    )
    def gather_kernel(table_hbm, idx_hbm, out_hbm,
                      idx_v0, idx_v1, rows0, rows1,
                      sem_g0, sem_g1, sem_o0, sem_o1):
        wid = lax.axis_index("s") * nc + lax.axis_index("c")
        row0 = wid * rows_per_w

        def idx_copy(g, buf):
            pltpu.sync_copy(
                idx_hbm.at[pl.ds(row0 + g * _ROWS, _ROWS)], buf)

        def streams(idx_buf, rows_buf, sem, go):
            op = pltpu.async_copy if go else pltpu.make_async_copy
            res = []
            for r in range(_ROWS):
                res.append(op(
                    table_hbm.at[idx_buf.at[r, pl.ds(0, _SPLIT)]],
                    rows_buf.at[r, pl.ds(0, _SPLIT)],
                    sem,
                ))
                res.append(op(
                    table_hbm.at[idx_buf.at[r, pl.ds(_SPLIT, H - _SPLIT)]],
                    rows_buf.at[r, pl.ds(_SPLIT, H - _SPLIT)],
                    sem,
                ))
            return res

        def fire_gathers(idx_buf, rows_buf, sem):
            streams(idx_buf, rows_buf, sem, True)

        def wait_gathers(idx_buf, rows_buf, sem):
            for d in streams(idx_buf, rows_buf, sem, False):
                d.wait()

        def fire_out(g, rows_buf, sem):
            pltpu.async_copy(
                rows_buf, out_hbm.at[pl.ds(row0 + g * _ROWS, _ROWS)], sem)

        def wait_out(rows_buf, sem):
            pltpu.make_async_copy(
                rows_buf, out_hbm.at[pl.ds(row0, _ROWS)], sem).wait()

        idx_copy(0, idx_v0)
        fire_gathers(idx_v0, rows0, sem_g0)
        idx_copy(1, idx_v1)

        def body(i, carry):
            g0 = 2 * i

            wait_gathers(idx_v0, rows0, sem_g0)

            @pl.when(i > 0)
            def _():
                wait_out(rows1, sem_o1)

            fire_gathers(idx_v1, rows1, sem_g1)
            fire_out(g0, rows0, sem_o0)
            idx_copy(jnp.minimum(g0 + 2, n_iters - 1), idx_v0)

            wait_gathers(idx_v1, rows1, sem_g1)
            wait_out(rows0, sem_o0)
            fire_gathers(idx_v0, rows0, sem_g0)
            fire_out(g0 + 1, rows1, sem_o1)
            idx_copy(jnp.minimum(g0 + 3, n_iters - 1), idx_v1)
            return carry

        lax.fori_loop(0, n_iters // 2, body, 0)
        wait_gathers(idx_v0, rows0, sem_g0)
        wait_out(rows1, sem_o1)

    return gather_kernel


def kernel(positions, pe):
    B, H = positions.shape
    V, D = pe.shape
    return _make_gather(B, H, D, 32)(pe, positions)
